# baseline (speedup 1.0000x reference)
import functools

import jax
import jax.numpy as jnp
from jax import lax
from jax.experimental import pallas as pl
from jax.experimental.pallas import tpu as pltpu
from jax.experimental.pallas import tpu_sc as plsc

_NW = 32
_L = 16
_LN2 = 0.6931471805599453
_C_SC = 0
_BI = 256


def _bilin(m):
    c0 = m[0, 0]
    c1 = m[1, 0] - m[0, 0]
    c2 = m[0, 1] - m[0, 0]
    c3 = m[1, 1] - m[1, 0] - m[0, 1] + m[0, 0]
    return jnp.stack([c0, c1, c2, c3])


def _rsqrt16(d2):
    i = lax.bitcast_convert_type(d2, jnp.int32)
    y = lax.bitcast_convert_type(jnp.int32(0x5F3759DF) - (i >> 1), jnp.float32)
    h = 0.5 * d2
    for _ in range(3):
        y = y * (1.5 - h * y * y)
    return y


def _ln16(b):
    i = lax.bitcast_convert_type(b, jnp.int32)
    ex = ((i >> 23) & 0xFF) - 127
    m = lax.bitcast_convert_type((i & 0x7FFFFF) | 0x3F800000, jnp.float32)
    r = (m - 1.0) / (m + 1.0)
    r2 = r * r
    p = 2.0 / 5.0 + r2 * (2.0 / 7.0)
    p = 2.0 / 3.0 + r2 * p
    lnm = r * (2.0 + r2 * p)
    return ex.astype(jnp.float32) * _LN2 + lnm


def _sc_pair_kernel(fx_hbm, fy_hbm, fz_hbm, sp_hbm, par_hbm,
                    ofx_hbm, ofy_hbm, ofz_hbm, ope_hbm,
                    xv, yv, zv, sv, pv, obuf):
    n = 4096
    rows = n // _NW
    wid = lax.axis_index("s") * 2 + lax.axis_index("c")
    base = wid * rows

    pltpu.sync_copy(fx_hbm, xv)
    pltpu.sync_copy(fy_hbm, yv)
    pltpu.sync_copy(fz_hbm, zv)
    pltpu.sync_copy(sp_hbm, sv)
    pltpu.sync_copy(par_hbm, pv)

    pvs = [pv[pl.ds(t * _L, _L)] for t in range(3)]

    def par(k):
        v = pvs[k // _L][k % _L]
        return jnp.full((_L,), v, jnp.float32)

    met = [par(t) for t in range(6)]
    pbc = [par(6 + a) for a in range(3)]
    cl = [[par(30 + 3 * a + b) for b in range(3)] for a in range(3)]
    cut2 = par(12)
    isc = [par(14 + t) for t in range(4)]
    acc_ = [par(18 + t) for t in range(4)]
    eoa = [par(22 + t) for t in range(4)]
    neos = [par(26 + t) for t in range(4)]

    lane = lax.iota(jnp.int32, _L)

    def group_body(g, _):
        ob = base + g * _L
        ox = xv[pl.ds(ob, _L)]
        oy = yv[pl.ds(ob, _L)]
        oz = zv[pl.ds(ob, _L)]
        osp = sv[pl.ds(ob, _L)]
        own_id = ob + lane
        is0 = isc[0] + isc[1] * osp
        is1 = isc[2] + isc[3] * osp
        a0 = acc_[0] + acc_[1] * osp
        a1 = acc_[2] + acc_[3] * osp
        ea0 = eoa[0] + eoa[1] * osp
        ea1 = eoa[2] + eoa[3] * osp
        es0 = neos[0] + neos[1] * osp
        es1 = neos[2] + neos[3] * osp

        def j_body(jb, acc):
            afx, afy, afz, ape = acc
            jx = xv[pl.ds(jb * _L, _L)]
            jy = yv[pl.ds(jb * _L, _L)]
            jz = zv[pl.ds(jb * _L, _L)]
            js = sv[pl.ds(jb * _L, _L)]
            for l in range(_L):
                jglob = jb * _L + l
                df = []
                for ovec, jvec in ((ox, jx), (oy, jy), (oz, jz)):
                    dd = jnp.full((_L,), jvec[l], jnp.float32) - ovec
                    w = (jnp.where(dd > 0.5, 1.0, 0.0)
                         - jnp.where(dd < -0.5, 1.0, 0.0))
                    df.append(dd - w * pbc[len(df)])
                d2 = (met[0] * df[0] * df[0] + met[1] * df[1] * df[1]
                      + met[2] * df[2] * df[2] + met[3] * df[0] * df[1]
                      + met[4] * df[0] * df[2] + met[5] * df[1] * df[2])
                y = _rsqrt16(jnp.maximum(d2, 1e-12))
                d = d2 * y
                sj = jnp.full((_L,), js[l], jnp.float32)
                inv_s = is0 + is1 * sj
                a = a0 + a1 * sj
                e_a = ea0 + ea1 * sj
                ne_s = es0 + es1 * sj
                braw = 1.0 - d * inv_s
                inside = (d2 < cut2) & (braw > 0.0) & (own_id != jglob)
                b = jnp.where(inside, braw, 0.5)
                lnb = _ln16(b)
                p = jnp.exp(a * lnb)
                q = p / b
                pe = jnp.where(inside, e_a * p, 0.0)
                cf = jnp.where(inside, ne_s * q * y, 0.0)
                afx = afx + cf * df[0]
                afy = afy + cf * df[1]
                afz = afz + cf * df[2]
                ape = ape + pe
            return afx, afy, afz, ape

        z = jnp.zeros((_L,), jnp.float32)
        afx, afy, afz, ape = lax.fori_loop(0, _C_SC // _L, j_body, (z, z, z, z))
        obuf[pl.ds(g * _L, _L)] = afx * cl[0][0] + afy * cl[1][0] + afz * cl[2][0]
        obuf[pl.ds(rows + g * _L, _L)] = afx * cl[0][1] + afy * cl[1][1] + afz * cl[2][1]
        obuf[pl.ds(2 * rows + g * _L, _L)] = afx * cl[0][2] + afy * cl[1][2] + afz * cl[2][2]
        obuf[pl.ds(3 * rows + g * _L, _L)] = ape
        return 0

    lax.fori_loop(0, rows // _L, group_body, 0)

    pltpu.sync_copy(obuf.at[pl.ds(0, rows)], ofx_hbm.at[pl.ds(base, rows)])
    pltpu.sync_copy(obuf.at[pl.ds(rows, rows)], ofy_hbm.at[pl.ds(base, rows)])
    pltpu.sync_copy(obuf.at[pl.ds(2 * rows, rows)], ofz_hbm.at[pl.ds(base, rows)])
    pltpu.sync_copy(obuf.at[pl.ds(3 * rows, rows)], ope_hbm.at[pl.ds(base, rows)])


def _tc_pair_kernel(params_ref, row_ref, col_ref, out_ref):
    nj = row_ref.shape[1]
    bi = col_ref.shape[0]
    pid = pl.program_id(0)

    met = [params_ref[t] for t in range(6)]
    pbc = [params_ref[6 + m] for m in range(3)]
    cl = [[params_ref[30 + 3 * a + b] for b in range(3)] for a in range(3)]
    cut2 = params_ref[12]
    isc = [params_ref[14 + t] for t in range(4)]
    ac = [params_ref[18 + t] for t in range(4)]
    eoa = [params_ref[22 + t] for t in range(4)]
    neos = [params_ref[26 + t] for t in range(4)]

    dfrac = []
    for m in range(3):
        fi = col_ref[:, m].reshape(bi, 1)
        fj = row_ref[m, :].reshape(1, nj)
        df = fj - fi
        df = df - jnp.round(df) * pbc[m]
        dfrac.append(df)

    d2 = (met[0] * dfrac[0] * dfrac[0] + met[1] * dfrac[1] * dfrac[1]
          + met[2] * dfrac[2] * dfrac[2] + met[3] * dfrac[0] * dfrac[1]
          + met[4] * dfrac[0] * dfrac[2] + met[5] * dfrac[1] * dfrac[2])

    i_glob = pid * bi + jax.lax.broadcasted_iota(jnp.int32, (bi, nj), 0)
    j_glob = _C_SC + jax.lax.broadcasted_iota(jnp.int32, (bi, nj), 1)
    eye = i_glob == j_glob

    safe_d2 = jnp.where(eye, 1.0, d2)
    inv_d = lax.rsqrt(safe_d2)
    d = safe_d2 * inv_d

    si = col_ref[:, 3].reshape(bi, 1)
    sj = row_ref[3, :].reshape(1, nj)
    sij = si * sj

    def bl(c):
        return c[0] + c[1] * si + c[2] * sj + c[3] * sij

    inv_s = bl(isc)
    a = bl(ac)
    e_a = bl(eoa)
    ne_s = bl(neos)

    braw = 1.0 - d * inv_s
    inside = (d2 < cut2) & (braw > 0.0) & jnp.logical_not(eye)
    b = jnp.where(inside, braw, 0.5)
    lnb = jnp.log(b)
    q = jnp.exp((a - 1.0) * lnb)
    p = q * b

    pe = jnp.where(inside, e_a * p, 0.0)
    coeff = jnp.where(inside, ne_s * q * inv_d, 0.0)

    gfx = jnp.sum(coeff * dfrac[0], axis=1).reshape(bi, 1)
    gfy = jnp.sum(coeff * dfrac[1], axis=1).reshape(bi, 1)
    gfz = jnp.sum(coeff * dfrac[2], axis=1).reshape(bi, 1)
    pes = jnp.sum(pe, axis=1).reshape(bi, 1)
    fx = gfx * cl[0][0] + gfy * cl[1][0] + gfz * cl[2][0]
    fy = gfx * cl[0][1] + gfy * cl[1][1] + gfz * cl[2][1]
    fz = gfx * cl[0][2] + gfy * cl[1][2] + gfz * cl[2][2]
    zeros = jnp.zeros((bi, 4), dtype=jnp.float32)
    out_ref[...] = jnp.concatenate([fx, fy, fz, pes, zeros], axis=1)


def _inv3(c):
    a00, a01, a02 = c[0, 0], c[0, 1], c[0, 2]
    a10, a11, a12 = c[1, 0], c[1, 1], c[1, 2]
    a20, a21, a22 = c[2, 0], c[2, 1], c[2, 2]
    c00 = a11 * a22 - a12 * a21
    c01 = a02 * a21 - a01 * a22
    c02 = a01 * a12 - a02 * a11
    c10 = a12 * a20 - a10 * a22
    c11 = a00 * a22 - a02 * a20
    c12 = a02 * a10 - a00 * a12
    c20 = a10 * a21 - a11 * a20
    c21 = a01 * a20 - a00 * a21
    c22 = a00 * a11 - a01 * a10
    det = a00 * c00 + a01 * c10 + a02 * c20
    adj = jnp.stack([jnp.stack([c00, c01, c02]),
                     jnp.stack([c10, c11, c12]),
                     jnp.stack([c20, c21, c22])])
    return adj / det


def kernel(positions, cell, pbc, species, sigma_matrix, epsilon_matrix, alpha_matrix, cutoff):
    n = positions.shape[0]
    inv_cell = _inv3(cell)
    frac = positions @ inv_cell
    spf = species.astype(jnp.float32)

    g = cell @ cell.T
    cf32 = cutoff.astype(jnp.float32)
    params = jnp.zeros((48,), jnp.float32)
    params = params.at[0:3].set(jnp.diag(g).astype(jnp.float32))
    params = params.at[3].set(2.0 * g[0, 1])
    params = params.at[4].set(2.0 * g[0, 2])
    params = params.at[5].set(2.0 * g[1, 2])
    params = params.at[6:9].set(pbc.astype(jnp.float32))
    params = params.at[12].set(cf32 * cf32)
    params = params.at[13].set(cf32)
    params = params.at[14:18].set(_bilin(1.0 / sigma_matrix))
    params = params.at[18:22].set(_bilin(alpha_matrix))
    params = params.at[22:26].set(_bilin(epsilon_matrix / alpha_matrix))
    params = params.at[26:30].set(_bilin(-epsilon_matrix / sigma_matrix))
    params = params.at[30:39].set(cell.reshape(9).astype(jnp.float32))


    col = jnp.concatenate(
        [frac, spf[:, None], jnp.zeros((n, 4), jnp.float32)], axis=1)
    row = col[_C_SC:].T

    grid = (n // _BI,)
    tc_out = pl.pallas_call(
        _tc_pair_kernel,
        grid=grid,
        in_specs=[
            pl.BlockSpec(memory_space=pltpu.SMEM),
            pl.BlockSpec((8, n - _C_SC), lambda i: (0, 0)),
            pl.BlockSpec((_BI, 8), lambda i: (i, 0)),
        ],
        out_specs=pl.BlockSpec((_BI, 8), lambda i: (i, 0)),
        out_shape=jax.ShapeDtypeStruct((n, 8), jnp.float32),
    )(params, row, col)

    forces = tc_out[:, :3]
    energy = 0.5 * jnp.sum(tc_out[:, 3])
    return energy, forces


# speedup vs baseline: 1.0875x; 1.0875x over previous
import functools

import jax
import jax.numpy as jnp
from jax import lax
from jax.experimental import pallas as pl
from jax.experimental.pallas import tpu as pltpu
from jax.experimental.pallas import tpu_sc as plsc

_NW = 32
_L = 16
_LN2 = 0.6931471805599453
_C_SC = 1024
_BI = 256
_BJ = 256


def _bilin(m):
    c0 = m[0, 0]
    c1 = m[1, 0] - m[0, 0]
    c2 = m[0, 1] - m[0, 0]
    c3 = m[1, 1] - m[1, 0] - m[0, 1] + m[0, 0]
    return jnp.stack([c0, c1, c2, c3])


def _rsqrt16(d2):
    i = lax.bitcast_convert_type(d2, jnp.int32)
    y = lax.bitcast_convert_type(jnp.int32(0x5F3759DF) - (i >> 1), jnp.float32)
    h = 0.5 * d2
    for _ in range(3):
        y = y * (1.5 - h * y * y)
    return y


def _ln16(b):
    i = lax.bitcast_convert_type(b, jnp.int32)
    ex = ((i >> 23) & 0xFF) - 127
    m = lax.bitcast_convert_type((i & 0x7FFFFF) | 0x3F800000, jnp.float32)
    r = (m - 1.0) / (m + 1.0)
    r2 = r * r
    p = 2.0 / 5.0 + r2 * (2.0 / 7.0)
    p = 2.0 / 3.0 + r2 * p
    lnm = r * (2.0 + r2 * p)
    return ex.astype(jnp.float32) * _LN2 + lnm


def _sc_pair_kernel(fx_hbm, fy_hbm, fz_hbm, sp_hbm, par_hbm,
                    ofx_hbm, ofy_hbm, ofz_hbm, ope_hbm,
                    xv, yv, zv, sv, pv, obuf):
    c = _C_SC
    rows = c // _NW
    wid = lax.axis_index("s") * 2 + lax.axis_index("c")
    base = wid * rows

    pltpu.sync_copy(fx_hbm.at[pl.ds(0, c)], xv)
    pltpu.sync_copy(fy_hbm.at[pl.ds(0, c)], yv)
    pltpu.sync_copy(fz_hbm.at[pl.ds(0, c)], zv)
    pltpu.sync_copy(sp_hbm.at[pl.ds(0, c)], sv)
    pltpu.sync_copy(par_hbm, pv)

    pvs = [pv[pl.ds(t * _L, _L)] for t in range(3)]

    def par(k):
        v = pvs[k // _L][k % _L]
        return jnp.full((_L,), v, jnp.float32)

    met = [par(t) for t in range(6)]
    pbc = [par(6 + a) for a in range(3)]
    cl = [[par(30 + 3 * a + b) for b in range(3)] for a in range(3)]
    cut2 = par(12)
    isc = [par(14 + t) for t in range(4)]
    acc_ = [par(18 + t) for t in range(4)]
    eoa = [par(22 + t) for t in range(4)]
    neos = [par(26 + t) for t in range(4)]

    lane = lax.iota(jnp.int32, _L)

    def group_body(g, _):
        ob = base + g * _L
        ox = xv[pl.ds(ob, _L)]
        oy = yv[pl.ds(ob, _L)]
        oz = zv[pl.ds(ob, _L)]
        osp = sv[pl.ds(ob, _L)]
        own_id = ob + lane
        is0 = isc[0] + isc[1] * osp
        is1 = isc[2] + isc[3] * osp
        a0 = acc_[0] + acc_[1] * osp
        a1 = acc_[2] + acc_[3] * osp
        ea0 = eoa[0] + eoa[1] * osp
        ea1 = eoa[2] + eoa[3] * osp
        es0 = neos[0] + neos[1] * osp
        es1 = neos[2] + neos[3] * osp

        def j_body(jb, acc):
            afx, afy, afz, ape = acc
            jx = xv[pl.ds(jb * _L, _L)]
            jy = yv[pl.ds(jb * _L, _L)]
            jz = zv[pl.ds(jb * _L, _L)]
            js = sv[pl.ds(jb * _L, _L)]
            for l in range(_L):
                jglob = jb * _L + l
                df = []
                for ovec, jvec in ((ox, jx), (oy, jy), (oz, jz)):
                    dd = jnp.full((_L,), jvec[l], jnp.float32) - ovec
                    w = (jnp.where(dd > 0.5, 1.0, 0.0)
                         - jnp.where(dd < -0.5, 1.0, 0.0))
                    df.append(dd - w * pbc[len(df)])
                d2 = (met[0] * df[0] * df[0] + met[1] * df[1] * df[1]
                      + met[2] * df[2] * df[2] + met[3] * df[0] * df[1]
                      + met[4] * df[0] * df[2] + met[5] * df[1] * df[2])
                y = _rsqrt16(jnp.maximum(d2, 1e-12))
                d = d2 * y
                sj = jnp.full((_L,), js[l], jnp.float32)
                inv_s = is0 + is1 * sj
                a = a0 + a1 * sj
                e_a = ea0 + ea1 * sj
                ne_s = es0 + es1 * sj
                braw = 1.0 - d * inv_s
                inside = (d2 < cut2) & (braw > 0.0) & (own_id != jglob)
                b = jnp.where(inside, braw, 0.5)
                lnb = _ln16(b)
                p = jnp.exp(a * lnb)
                q = p / b
                pe = jnp.where(inside, e_a * p, 0.0)
                cf = jnp.where(inside, ne_s * q * y, 0.0)
                afx = afx + cf * df[0]
                afy = afy + cf * df[1]
                afz = afz + cf * df[2]
                ape = ape + pe
            return afx, afy, afz, ape

        z = jnp.zeros((_L,), jnp.float32)
        afx, afy, afz, ape = lax.fori_loop(0, c // _L, j_body, (z, z, z, z))
        obuf[pl.ds(g * _L, _L)] = afx * cl[0][0] + afy * cl[1][0] + afz * cl[2][0]
        obuf[pl.ds(rows + g * _L, _L)] = afx * cl[0][1] + afy * cl[1][1] + afz * cl[2][1]
        obuf[pl.ds(2 * rows + g * _L, _L)] = afx * cl[0][2] + afy * cl[1][2] + afz * cl[2][2]
        obuf[pl.ds(3 * rows + g * _L, _L)] = ape
        return 0

    lax.fori_loop(0, rows // _L, group_body, 0)

    pltpu.sync_copy(obuf.at[pl.ds(0, rows)], ofx_hbm.at[pl.ds(base, rows)])
    pltpu.sync_copy(obuf.at[pl.ds(rows, rows)], ofy_hbm.at[pl.ds(base, rows)])
    pltpu.sync_copy(obuf.at[pl.ds(2 * rows, rows)], ofz_hbm.at[pl.ds(base, rows)])
    pltpu.sync_copy(obuf.at[pl.ds(3 * rows, rows)], ope_hbm.at[pl.ds(base, rows)])


def _tc_pair_kernel(params_ref, row_ref, col_ref, orow_ref, ocol_ref):
    bj = row_ref.shape[1]
    bi = col_ref.shape[0]
    ib = pl.program_id(0)
    jb = pl.program_id(1)

    met = [params_ref[t] for t in range(6)]
    pbc = [params_ref[6 + m] for m in range(3)]
    cut2 = params_ref[12]
    isc = [params_ref[14 + t] for t in range(4)]
    ac = [params_ref[18 + t] for t in range(4)]
    eoa = [params_ref[22 + t] for t in range(4)]
    neos = [params_ref[26 + t] for t in range(4)]
    cl = [[params_ref[30 + 3 * a + b] for b in range(3)] for a in range(3)]

    @pl.when(jb == 0)
    def _():
        orow_ref[...] = jnp.zeros_like(orow_ref)

    include = ib * bi < _C_SC + (jb + 1) * bj

    @pl.when(include)
    def _():
        dfrac = []
        for m in range(3):
            fi = col_ref[:, m].reshape(bi, 1)
            fj = row_ref[m, :].reshape(1, bj)
            df = fj - fi
            df = df - jnp.round(df) * pbc[m]
            dfrac.append(df)

        d2 = (met[0] * dfrac[0] * dfrac[0] + met[1] * dfrac[1] * dfrac[1]
              + met[2] * dfrac[2] * dfrac[2] + met[3] * dfrac[0] * dfrac[1]
              + met[4] * dfrac[0] * dfrac[2] + met[5] * dfrac[1] * dfrac[2])

        i_glob = ib * bi + jax.lax.broadcasted_iota(jnp.int32, (bi, bj), 0)
        j_glob = (_C_SC + jb * bj
                  + jax.lax.broadcasted_iota(jnp.int32, (bi, bj), 1))
        tri = j_glob > i_glob

        safe_d2 = jnp.where(tri, d2, 1.0)
        inv_d = lax.rsqrt(safe_d2)
        d = safe_d2 * inv_d

        si = col_ref[:, 3].reshape(bi, 1)
        sj = row_ref[3, :].reshape(1, bj)
        sij = si * sj

        def bl(cc):
            return cc[0] + cc[1] * si + cc[2] * sj + cc[3] * sij

        inv_s = bl(isc)
        a = bl(ac)
        e_a = bl(eoa)
        ne_s = bl(neos)

        braw = 1.0 - d * inv_s
        inside = (d2 < cut2) & (braw > 0.0) & tri
        b = jnp.where(inside, braw, 0.5)
        lnb = jnp.log(b)
        q = jnp.exp((a - 1.0) * lnb)
        p = q * b

        pe = jnp.where(inside, e_a * p, 0.0)
        coeff = jnp.where(inside, ne_s * q * inv_d, 0.0)
        w0 = coeff * dfrac[0]
        w1 = coeff * dfrac[1]
        w2 = coeff * dfrac[2]

        gfx = jnp.sum(w0, axis=1).reshape(bi, 1)
        gfy = jnp.sum(w1, axis=1).reshape(bi, 1)
        gfz = jnp.sum(w2, axis=1).reshape(bi, 1)
        pes = jnp.sum(pe, axis=1).reshape(bi, 1)
        fx = gfx * cl[0][0] + gfy * cl[1][0] + gfz * cl[2][0]
        fy = gfx * cl[0][1] + gfy * cl[1][1] + gfz * cl[2][1]
        fz = gfx * cl[0][2] + gfy * cl[1][2] + gfz * cl[2][2]
        zeros = jnp.zeros((bi, 4), dtype=jnp.float32)
        orow_ref[...] += jnp.concatenate([fx, fy, fz, pes, zeros], axis=1)

        cgx = -jnp.sum(w0, axis=0).reshape(1, bj)
        cgy = -jnp.sum(w1, axis=0).reshape(1, bj)
        cgz = -jnp.sum(w2, axis=0).reshape(1, bj)
        cfx = cgx * cl[0][0] + cgy * cl[1][0] + cgz * cl[2][0]
        cfy = cgx * cl[0][1] + cgy * cl[1][1] + cgz * cl[2][1]
        cfz = cgx * cl[0][2] + cgy * cl[1][2] + cgz * cl[2][2]
        zcol = jnp.zeros((5, bj), dtype=jnp.float32)
        ocol_ref[...] = jnp.concatenate(
            [cfx, cfy, cfz, zcol], axis=0)[None]

    @pl.when(jnp.logical_not(include))
    def _():
        ocol_ref[...] = jnp.zeros_like(ocol_ref)


def _inv3(c):
    a00, a01, a02 = c[0, 0], c[0, 1], c[0, 2]
    a10, a11, a12 = c[1, 0], c[1, 1], c[1, 2]
    a20, a21, a22 = c[2, 0], c[2, 1], c[2, 2]
    c00 = a11 * a22 - a12 * a21
    c01 = a02 * a21 - a01 * a22
    c02 = a01 * a12 - a02 * a11
    c10 = a12 * a20 - a10 * a22
    c11 = a00 * a22 - a02 * a20
    c12 = a02 * a10 - a00 * a12
    c20 = a10 * a21 - a11 * a20
    c21 = a01 * a20 - a00 * a21
    c22 = a00 * a11 - a01 * a10
    det = a00 * c00 + a01 * c10 + a02 * c20
    adj = jnp.stack([jnp.stack([c00, c01, c02]),
                     jnp.stack([c10, c11, c12]),
                     jnp.stack([c20, c21, c22])])
    return adj / det


def kernel(positions, cell, pbc, species, sigma_matrix, epsilon_matrix, alpha_matrix, cutoff):
    n = positions.shape[0]
    inv_cell = _inv3(cell)
    frac = positions @ inv_cell
    spf = species.astype(jnp.float32)

    g = cell @ cell.T
    cf32 = cutoff.astype(jnp.float32)
    params = jnp.zeros((48,), jnp.float32)
    params = params.at[0:3].set(jnp.diag(g).astype(jnp.float32))
    params = params.at[3].set(2.0 * g[0, 1])
    params = params.at[4].set(2.0 * g[0, 2])
    params = params.at[5].set(2.0 * g[1, 2])
    params = params.at[6:9].set(pbc.astype(jnp.float32))
    params = params.at[12].set(cf32 * cf32)
    params = params.at[14:18].set(_bilin(1.0 / sigma_matrix))
    params = params.at[18:22].set(_bilin(alpha_matrix))
    params = params.at[22:26].set(_bilin(epsilon_matrix / alpha_matrix))
    params = params.at[26:30].set(_bilin(-epsilon_matrix / sigma_matrix))
    params = params.at[30:39].set(cell.reshape(9).astype(jnp.float32))

    mesh = plsc.VectorSubcoreMesh(core_axis_name="c", subcore_axis_name="s")
    sc_f = functools.partial(
        pl.kernel,
        mesh=mesh,
        out_type=[jax.ShapeDtypeStruct((_C_SC,), jnp.float32)] * 4,
        scratch_types=[
            pltpu.VMEM((_C_SC,), jnp.float32),
            pltpu.VMEM((_C_SC,), jnp.float32),
            pltpu.VMEM((_C_SC,), jnp.float32),
            pltpu.VMEM((_C_SC,), jnp.float32),
            pltpu.VMEM((48,), jnp.float32),
            pltpu.VMEM((4 * _C_SC // _NW,), jnp.float32),
        ],
    )(_sc_pair_kernel)
    ofx, ofy, ofz, ope = sc_f(frac[:, 0], frac[:, 1], frac[:, 2], spf, params)

    col = jnp.concatenate(
        [frac, spf[:, None], jnp.zeros((n, 4), jnp.float32)], axis=1)
    row = col[_C_SC:].T

    njb = (n - _C_SC) // _BJ
    nib = n // _BI
    orow, ocol = pl.pallas_call(
        _tc_pair_kernel,
        grid=(nib, njb),
        in_specs=[
            pl.BlockSpec(memory_space=pltpu.SMEM),
            pl.BlockSpec((8, _BJ), lambda i, j: (0, j)),
            pl.BlockSpec((_BI, 8), lambda i, j: (i, 0)),
        ],
        out_specs=[
            pl.BlockSpec((_BI, 8), lambda i, j: (i, 0)),
            pl.BlockSpec((1, 8, _BJ), lambda i, j: (i, 0, j)),
        ],
        out_shape=[
            jax.ShapeDtypeStruct((n, 8), jnp.float32),
            jax.ShapeDtypeStruct((nib, 8, n - _C_SC), jnp.float32),
        ],
    )(params, row, col)

    col_sum = jnp.sum(ocol, axis=0)
    sc_force = jnp.stack([ofx, ofy, ofz], axis=1)
    tail = jnp.concatenate([sc_force, col_sum[:3].T], axis=0)
    forces = orow[:, :3] + tail
    energy = 0.5 * jnp.sum(ope) + jnp.sum(orow[:, 3])
    return energy, forces


# speedup vs baseline: 1.3272x; 1.2204x over previous
import functools

import jax
import jax.numpy as jnp
from jax import lax
from jax.experimental import pallas as pl
from jax.experimental.pallas import tpu as pltpu
from jax.experimental.pallas import tpu_sc as plsc

_NW = 32
_L = 16
_LN2 = 0.6931471805599453
_C_SC = 1024
_BI = 512
_BJ = 512


def _bilin(m):
    c0 = m[0, 0]
    c1 = m[1, 0] - m[0, 0]
    c2 = m[0, 1] - m[0, 0]
    c3 = m[1, 1] - m[1, 0] - m[0, 1] + m[0, 0]
    return jnp.stack([c0, c1, c2, c3])


def _rsqrt16(d2):
    i = lax.bitcast_convert_type(d2, jnp.int32)
    y = lax.bitcast_convert_type(jnp.int32(0x5F3759DF) - (i >> 1), jnp.float32)
    h = 0.5 * d2
    for _ in range(3):
        y = y * (1.5 - h * y * y)
    return y


def _ln16(b):
    i = lax.bitcast_convert_type(b, jnp.int32)
    ex = ((i >> 23) & 0xFF) - 127
    m = lax.bitcast_convert_type((i & 0x7FFFFF) | 0x3F800000, jnp.float32)
    r = (m - 1.0) / (m + 1.0)
    r2 = r * r
    p = 2.0 / 5.0 + r2 * (2.0 / 7.0)
    p = 2.0 / 3.0 + r2 * p
    lnm = r * (2.0 + r2 * p)
    return ex.astype(jnp.float32) * _LN2 + lnm


def _sc_pair_kernel(fx_hbm, fy_hbm, fz_hbm, sp_hbm, par_hbm,
                    ofx_hbm, ofy_hbm, ofz_hbm, ope_hbm,
                    xv, yv, zv, sv, pv, obuf):
    c = _C_SC
    rows = c // _NW
    wid = lax.axis_index("s") * 2 + lax.axis_index("c")
    base = wid * rows

    pltpu.sync_copy(fx_hbm.at[pl.ds(0, c)], xv)
    pltpu.sync_copy(fy_hbm.at[pl.ds(0, c)], yv)
    pltpu.sync_copy(fz_hbm.at[pl.ds(0, c)], zv)
    pltpu.sync_copy(sp_hbm.at[pl.ds(0, c)], sv)
    pltpu.sync_copy(par_hbm, pv)

    pvs = [pv[pl.ds(t * _L, _L)] for t in range(3)]

    def par(k):
        v = pvs[k // _L][k % _L]
        return jnp.full((_L,), v, jnp.float32)

    met = [par(t) for t in range(6)]
    pbc = [par(6 + a) for a in range(3)]
    cl = [[par(30 + 3 * a + b) for b in range(3)] for a in range(3)]
    cut2 = par(12)
    isc = [par(14 + t) for t in range(4)]
    acc_ = [par(18 + t) for t in range(4)]
    eoa = [par(22 + t) for t in range(4)]
    neos = [par(26 + t) for t in range(4)]

    lane = lax.iota(jnp.int32, _L)

    def group_body(g, _):
        ob = base + g * _L
        ox = xv[pl.ds(ob, _L)]
        oy = yv[pl.ds(ob, _L)]
        oz = zv[pl.ds(ob, _L)]
        osp = sv[pl.ds(ob, _L)]
        own_id = ob + lane
        is0 = isc[0] + isc[1] * osp
        is1 = isc[2] + isc[3] * osp
        a0 = acc_[0] + acc_[1] * osp
        a1 = acc_[2] + acc_[3] * osp
        ea0 = eoa[0] + eoa[1] * osp
        ea1 = eoa[2] + eoa[3] * osp
        es0 = neos[0] + neos[1] * osp
        es1 = neos[2] + neos[3] * osp

        def j_body(jb, acc):
            afx, afy, afz, ape = acc
            jx = xv[pl.ds(jb * _L, _L)]
            jy = yv[pl.ds(jb * _L, _L)]
            jz = zv[pl.ds(jb * _L, _L)]
            js = sv[pl.ds(jb * _L, _L)]
            for l in range(_L):
                jglob = jb * _L + l
                df = []
                for ovec, jvec in ((ox, jx), (oy, jy), (oz, jz)):
                    dd = jnp.full((_L,), jvec[l], jnp.float32) - ovec
                    w = (jnp.where(dd > 0.5, 1.0, 0.0)
                         - jnp.where(dd < -0.5, 1.0, 0.0))
                    df.append(dd - w * pbc[len(df)])
                d2 = (met[0] * df[0] * df[0] + met[1] * df[1] * df[1]
                      + met[2] * df[2] * df[2] + met[3] * df[0] * df[1]
                      + met[4] * df[0] * df[2] + met[5] * df[1] * df[2])
                y = _rsqrt16(jnp.maximum(d2, 1e-12))
                d = d2 * y
                sj = jnp.full((_L,), js[l], jnp.float32)
                inv_s = is0 + is1 * sj
                a = a0 + a1 * sj
                e_a = ea0 + ea1 * sj
                ne_s = es0 + es1 * sj
                braw = 1.0 - d * inv_s
                inside = (d2 < cut2) & (braw > 0.0) & (own_id != jglob)
                b = jnp.where(inside, braw, 0.5)
                lnb = _ln16(b)
                p = jnp.exp(a * lnb)
                q = p / b
                pe = jnp.where(inside, e_a * p, 0.0)
                cf = jnp.where(inside, ne_s * q * y, 0.0)
                afx = afx + cf * df[0]
                afy = afy + cf * df[1]
                afz = afz + cf * df[2]
                ape = ape + pe
            return afx, afy, afz, ape

        z = jnp.zeros((_L,), jnp.float32)
        afx, afy, afz, ape = lax.fori_loop(0, c // _L, j_body, (z, z, z, z))
        obuf[pl.ds(g * _L, _L)] = afx * cl[0][0] + afy * cl[1][0] + afz * cl[2][0]
        obuf[pl.ds(rows + g * _L, _L)] = afx * cl[0][1] + afy * cl[1][1] + afz * cl[2][1]
        obuf[pl.ds(2 * rows + g * _L, _L)] = afx * cl[0][2] + afy * cl[1][2] + afz * cl[2][2]
        obuf[pl.ds(3 * rows + g * _L, _L)] = ape
        return 0

    lax.fori_loop(0, rows // _L, group_body, 0)

    pltpu.sync_copy(obuf.at[pl.ds(0, rows)], ofx_hbm.at[pl.ds(base, rows)])
    pltpu.sync_copy(obuf.at[pl.ds(rows, rows)], ofy_hbm.at[pl.ds(base, rows)])
    pltpu.sync_copy(obuf.at[pl.ds(2 * rows, rows)], ofz_hbm.at[pl.ds(base, rows)])
    pltpu.sync_copy(obuf.at[pl.ds(3 * rows, rows)], ope_hbm.at[pl.ds(base, rows)])


def _tc_pair_kernel(params_ref, row_ref, col_ref, orow_ref, ocol_ref):
    bj = row_ref.shape[1]
    bi = col_ref.shape[0]
    ib = pl.program_id(0)
    jb = pl.program_id(1)

    met = [params_ref[t] for t in range(6)]
    pbc = [params_ref[6 + m] for m in range(3)]
    cut2 = params_ref[12]
    isc = [params_ref[14 + t] for t in range(4)]
    ac = [params_ref[18 + t] for t in range(4)]
    eoa = [params_ref[22 + t] for t in range(4)]
    neos = [params_ref[26 + t] for t in range(4)]
    cl = [[params_ref[30 + 3 * a + b] for b in range(3)] for a in range(3)]

    @pl.when(jb == 0)
    def _():
        orow_ref[...] = jnp.zeros_like(orow_ref)

    include = ib * bi < _C_SC + (jb + 1) * bj

    @pl.when(include)
    def _():
        dfrac = []
        for m in range(3):
            fi = col_ref[:, m].reshape(bi, 1)
            fj = row_ref[m, :].reshape(1, bj)
            df = fj - fi
            df = df - jnp.round(df) * pbc[m]
            dfrac.append(df)

        d2 = (met[0] * dfrac[0] * dfrac[0] + met[1] * dfrac[1] * dfrac[1]
              + met[2] * dfrac[2] * dfrac[2] + met[3] * dfrac[0] * dfrac[1]
              + met[4] * dfrac[0] * dfrac[2] + met[5] * dfrac[1] * dfrac[2])

        i_glob = ib * bi + jax.lax.broadcasted_iota(jnp.int32, (bi, bj), 0)
        j_glob = (_C_SC + jb * bj
                  + jax.lax.broadcasted_iota(jnp.int32, (bi, bj), 1))
        tri = j_glob > i_glob

        safe_d2 = jnp.where(tri, d2, 1.0)
        inv_d = lax.rsqrt(safe_d2)
        d = safe_d2 * inv_d

        si = col_ref[:, 3].reshape(bi, 1)
        sj = row_ref[3, :].reshape(1, bj)
        sij = si * sj

        def bl(cc):
            return cc[0] + cc[1] * si + cc[2] * sj + cc[3] * sij

        inv_s = bl(isc)
        a = bl(ac)
        e_a = bl(eoa)
        ne_s = bl(neos)

        braw = 1.0 - d * inv_s
        inside = (d2 < cut2) & (braw > 0.0) & tri
        b = jnp.where(inside, braw, 0.5)
        lnb = jnp.log(b)
        q = jnp.exp((a - 1.0) * lnb)
        p = q * b

        pe = jnp.where(inside, e_a * p, 0.0)
        coeff = jnp.where(inside, ne_s * q * inv_d, 0.0)
        w0 = coeff * dfrac[0]
        w1 = coeff * dfrac[1]
        w2 = coeff * dfrac[2]

        gfx = jnp.sum(w0, axis=1).reshape(bi, 1)
        gfy = jnp.sum(w1, axis=1).reshape(bi, 1)
        gfz = jnp.sum(w2, axis=1).reshape(bi, 1)
        pes = jnp.sum(pe, axis=1).reshape(bi, 1)
        fx = gfx * cl[0][0] + gfy * cl[1][0] + gfz * cl[2][0]
        fy = gfx * cl[0][1] + gfy * cl[1][1] + gfz * cl[2][1]
        fz = gfx * cl[0][2] + gfy * cl[1][2] + gfz * cl[2][2]
        zeros = jnp.zeros((bi, 4), dtype=jnp.float32)
        orow_ref[...] += jnp.concatenate([fx, fy, fz, pes, zeros], axis=1)

        cgx = -jnp.sum(w0, axis=0).reshape(1, bj)
        cgy = -jnp.sum(w1, axis=0).reshape(1, bj)
        cgz = -jnp.sum(w2, axis=0).reshape(1, bj)
        cfx = cgx * cl[0][0] + cgy * cl[1][0] + cgz * cl[2][0]
        cfy = cgx * cl[0][1] + cgy * cl[1][1] + cgz * cl[2][1]
        cfz = cgx * cl[0][2] + cgy * cl[1][2] + cgz * cl[2][2]
        zcol = jnp.zeros((5, bj), dtype=jnp.float32)
        ocol_ref[...] = jnp.concatenate(
            [cfx, cfy, cfz, zcol], axis=0)[None]

    @pl.when(jnp.logical_not(include))
    def _():
        ocol_ref[...] = jnp.zeros_like(ocol_ref)


def _inv3(c):
    a00, a01, a02 = c[0, 0], c[0, 1], c[0, 2]
    a10, a11, a12 = c[1, 0], c[1, 1], c[1, 2]
    a20, a21, a22 = c[2, 0], c[2, 1], c[2, 2]
    c00 = a11 * a22 - a12 * a21
    c01 = a02 * a21 - a01 * a22
    c02 = a01 * a12 - a02 * a11
    c10 = a12 * a20 - a10 * a22
    c11 = a00 * a22 - a02 * a20
    c12 = a02 * a10 - a00 * a12
    c20 = a10 * a21 - a11 * a20
    c21 = a01 * a20 - a00 * a21
    c22 = a00 * a11 - a01 * a10
    det = a00 * c00 + a01 * c10 + a02 * c20
    adj = jnp.stack([jnp.stack([c00, c01, c02]),
                     jnp.stack([c10, c11, c12]),
                     jnp.stack([c20, c21, c22])])
    return adj / det


def kernel(positions, cell, pbc, species, sigma_matrix, epsilon_matrix, alpha_matrix, cutoff):
    n = positions.shape[0]
    inv_cell = _inv3(cell)
    frac = positions @ inv_cell
    spf = species.astype(jnp.float32)

    g = cell @ cell.T
    cf32 = cutoff.astype(jnp.float32)
    params = jnp.zeros((48,), jnp.float32)
    params = params.at[0:3].set(jnp.diag(g).astype(jnp.float32))
    params = params.at[3].set(2.0 * g[0, 1])
    params = params.at[4].set(2.0 * g[0, 2])
    params = params.at[5].set(2.0 * g[1, 2])
    params = params.at[6:9].set(pbc.astype(jnp.float32))
    params = params.at[12].set(cf32 * cf32)
    params = params.at[14:18].set(_bilin(1.0 / sigma_matrix))
    params = params.at[18:22].set(_bilin(alpha_matrix))
    params = params.at[22:26].set(_bilin(epsilon_matrix / alpha_matrix))
    params = params.at[26:30].set(_bilin(-epsilon_matrix / sigma_matrix))
    params = params.at[30:39].set(cell.reshape(9).astype(jnp.float32))

    mesh = plsc.VectorSubcoreMesh(core_axis_name="c", subcore_axis_name="s")
    sc_f = functools.partial(
        pl.kernel,
        mesh=mesh,
        out_type=[jax.ShapeDtypeStruct((_C_SC,), jnp.float32)] * 4,
        scratch_types=[
            pltpu.VMEM((_C_SC,), jnp.float32),
            pltpu.VMEM((_C_SC,), jnp.float32),
            pltpu.VMEM((_C_SC,), jnp.float32),
            pltpu.VMEM((_C_SC,), jnp.float32),
            pltpu.VMEM((48,), jnp.float32),
            pltpu.VMEM((4 * _C_SC // _NW,), jnp.float32),
        ],
    )(_sc_pair_kernel)
    ofx, ofy, ofz, ope = sc_f(frac[:, 0], frac[:, 1], frac[:, 2], spf, params)

    col = jnp.concatenate(
        [frac, spf[:, None], jnp.zeros((n, 4), jnp.float32)], axis=1)
    row = col[_C_SC:].T

    njb = (n - _C_SC) // _BJ
    nib = n // _BI
    orow, ocol = pl.pallas_call(
        _tc_pair_kernel,
        grid=(nib, njb),
        in_specs=[
            pl.BlockSpec(memory_space=pltpu.SMEM),
            pl.BlockSpec((8, _BJ), lambda i, j: (0, j)),
            pl.BlockSpec((_BI, 8), lambda i, j: (i, 0)),
        ],
        out_specs=[
            pl.BlockSpec((_BI, 8), lambda i, j: (i, 0)),
            pl.BlockSpec((1, 8, _BJ), lambda i, j: (i, 0, j)),
        ],
        out_shape=[
            jax.ShapeDtypeStruct((n, 8), jnp.float32),
            jax.ShapeDtypeStruct((nib, 8, n - _C_SC), jnp.float32),
        ],
    )(params, row, col)

    col_sum = jnp.sum(ocol, axis=0)
    sc_force = jnp.stack([ofx, ofy, ofz], axis=1)
    tail = jnp.concatenate([sc_force, col_sum[:3].T], axis=0)
    forces = orow[:, :3] + tail
    energy = 0.5 * jnp.sum(ope) + jnp.sum(orow[:, 3])
    return energy, forces


# speedup vs baseline: 1.3762x; 1.0369x over previous
import functools

import jax
import jax.numpy as jnp
from jax import lax
from jax.experimental import pallas as pl
from jax.experimental.pallas import tpu as pltpu
from jax.experimental.pallas import tpu_sc as plsc

_NW = 32
_L = 16
_LN2 = 0.6931471805599453
_C_SC = 1024
_BI = 1024
_BJ = 1024


def _bilin(m):
    c0 = m[0, 0]
    c1 = m[1, 0] - m[0, 0]
    c2 = m[0, 1] - m[0, 0]
    c3 = m[1, 1] - m[1, 0] - m[0, 1] + m[0, 0]
    return jnp.stack([c0, c1, c2, c3])


def _rsqrt16(d2):
    i = lax.bitcast_convert_type(d2, jnp.int32)
    y = lax.bitcast_convert_type(jnp.int32(0x5F3759DF) - (i >> 1), jnp.float32)
    h = 0.5 * d2
    for _ in range(3):
        y = y * (1.5 - h * y * y)
    return y


def _ln16(b):
    i = lax.bitcast_convert_type(b, jnp.int32)
    ex = ((i >> 23) & 0xFF) - 127
    m = lax.bitcast_convert_type((i & 0x7FFFFF) | 0x3F800000, jnp.float32)
    r = (m - 1.0) / (m + 1.0)
    r2 = r * r
    p = 2.0 / 5.0 + r2 * (2.0 / 7.0)
    p = 2.0 / 3.0 + r2 * p
    lnm = r * (2.0 + r2 * p)
    return ex.astype(jnp.float32) * _LN2 + lnm


def _sc_pair_kernel(fx_hbm, fy_hbm, fz_hbm, sp_hbm, par_hbm,
                    ofx_hbm, ofy_hbm, ofz_hbm, ope_hbm,
                    xv, yv, zv, sv, pv, obuf):
    c = _C_SC
    rows = c // _NW
    wid = lax.axis_index("s") * 2 + lax.axis_index("c")
    base = wid * rows

    pltpu.sync_copy(fx_hbm.at[pl.ds(0, c)], xv)
    pltpu.sync_copy(fy_hbm.at[pl.ds(0, c)], yv)
    pltpu.sync_copy(fz_hbm.at[pl.ds(0, c)], zv)
    pltpu.sync_copy(sp_hbm.at[pl.ds(0, c)], sv)
    pltpu.sync_copy(par_hbm, pv)

    pvs = [pv[pl.ds(t * _L, _L)] for t in range(3)]

    def par(k):
        v = pvs[k // _L][k % _L]
        return jnp.full((_L,), v, jnp.float32)

    met = [par(t) for t in range(6)]
    pbc = [par(6 + a) for a in range(3)]
    cl = [[par(30 + 3 * a + b) for b in range(3)] for a in range(3)]
    cut2 = par(12)
    isc = [par(14 + t) for t in range(4)]
    acc_ = [par(18 + t) for t in range(4)]
    eoa = [par(22 + t) for t in range(4)]
    neos = [par(26 + t) for t in range(4)]

    lane = lax.iota(jnp.int32, _L)

    def group_body(g, _):
        ob = base + g * _L
        ox = xv[pl.ds(ob, _L)]
        oy = yv[pl.ds(ob, _L)]
        oz = zv[pl.ds(ob, _L)]
        osp = sv[pl.ds(ob, _L)]
        own_id = ob + lane
        is0 = isc[0] + isc[1] * osp
        is1 = isc[2] + isc[3] * osp
        a0 = acc_[0] + acc_[1] * osp
        a1 = acc_[2] + acc_[3] * osp
        ea0 = eoa[0] + eoa[1] * osp
        ea1 = eoa[2] + eoa[3] * osp
        es0 = neos[0] + neos[1] * osp
        es1 = neos[2] + neos[3] * osp

        def j_body(jb, acc):
            afx, afy, afz, ape = acc
            jx = xv[pl.ds(jb * _L, _L)]
            jy = yv[pl.ds(jb * _L, _L)]
            jz = zv[pl.ds(jb * _L, _L)]
            js = sv[pl.ds(jb * _L, _L)]
            for l in range(_L):
                jglob = jb * _L + l
                df = []
                for ovec, jvec in ((ox, jx), (oy, jy), (oz, jz)):
                    dd = jnp.full((_L,), jvec[l], jnp.float32) - ovec
                    w = (jnp.where(dd > 0.5, 1.0, 0.0)
                         - jnp.where(dd < -0.5, 1.0, 0.0))
                    df.append(dd - w * pbc[len(df)])
                d2 = (met[0] * df[0] * df[0] + met[1] * df[1] * df[1]
                      + met[2] * df[2] * df[2] + met[3] * df[0] * df[1]
                      + met[4] * df[0] * df[2] + met[5] * df[1] * df[2])
                y = _rsqrt16(jnp.maximum(d2, 1e-12))
                d = d2 * y
                sj = jnp.full((_L,), js[l], jnp.float32)
                inv_s = is0 + is1 * sj
                a = a0 + a1 * sj
                e_a = ea0 + ea1 * sj
                ne_s = es0 + es1 * sj
                braw = 1.0 - d * inv_s
                inside = (d2 < cut2) & (braw > 0.0) & (own_id != jglob)
                b = jnp.where(inside, braw, 0.5)
                lnb = _ln16(b)
                p = jnp.exp(a * lnb)
                q = p / b
                pe = jnp.where(inside, e_a * p, 0.0)
                cf = jnp.where(inside, ne_s * q * y, 0.0)
                afx = afx + cf * df[0]
                afy = afy + cf * df[1]
                afz = afz + cf * df[2]
                ape = ape + pe
            return afx, afy, afz, ape

        z = jnp.zeros((_L,), jnp.float32)
        afx, afy, afz, ape = lax.fori_loop(0, c // _L, j_body, (z, z, z, z))
        obuf[pl.ds(g * _L, _L)] = afx * cl[0][0] + afy * cl[1][0] + afz * cl[2][0]
        obuf[pl.ds(rows + g * _L, _L)] = afx * cl[0][1] + afy * cl[1][1] + afz * cl[2][1]
        obuf[pl.ds(2 * rows + g * _L, _L)] = afx * cl[0][2] + afy * cl[1][2] + afz * cl[2][2]
        obuf[pl.ds(3 * rows + g * _L, _L)] = ape
        return 0

    lax.fori_loop(0, rows // _L, group_body, 0)

    pltpu.sync_copy(obuf.at[pl.ds(0, rows)], ofx_hbm.at[pl.ds(base, rows)])
    pltpu.sync_copy(obuf.at[pl.ds(rows, rows)], ofy_hbm.at[pl.ds(base, rows)])
    pltpu.sync_copy(obuf.at[pl.ds(2 * rows, rows)], ofz_hbm.at[pl.ds(base, rows)])
    pltpu.sync_copy(obuf.at[pl.ds(3 * rows, rows)], ope_hbm.at[pl.ds(base, rows)])


def _tc_pair_kernel(params_ref, row_ref, col_ref, orow_ref, ocol_ref):
    bj = row_ref.shape[1]
    bi = col_ref.shape[0]
    ib = pl.program_id(0)
    jb = pl.program_id(1)

    met = [params_ref[t] for t in range(6)]
    pbc = [params_ref[6 + m] for m in range(3)]
    cut2 = params_ref[12]
    isc = [params_ref[14 + t] for t in range(4)]
    ac = [params_ref[18 + t] for t in range(4)]
    eoa = [params_ref[22 + t] for t in range(4)]
    neos = [params_ref[26 + t] for t in range(4)]
    cl = [[params_ref[30 + 3 * a + b] for b in range(3)] for a in range(3)]

    @pl.when(jb == 0)
    def _():
        orow_ref[...] = jnp.zeros_like(orow_ref)

    include = ib * bi < _C_SC + (jb + 1) * bj

    @pl.when(include)
    def _():
        dfrac = []
        for m in range(3):
            fi = col_ref[:, m].reshape(bi, 1)
            fj = row_ref[m, :].reshape(1, bj)
            df = fj - fi
            df = df - jnp.round(df) * pbc[m]
            dfrac.append(df)

        d2 = (met[0] * dfrac[0] * dfrac[0] + met[1] * dfrac[1] * dfrac[1]
              + met[2] * dfrac[2] * dfrac[2] + met[3] * dfrac[0] * dfrac[1]
              + met[4] * dfrac[0] * dfrac[2] + met[5] * dfrac[1] * dfrac[2])

        i_glob = ib * bi + jax.lax.broadcasted_iota(jnp.int32, (bi, bj), 0)
        j_glob = (_C_SC + jb * bj
                  + jax.lax.broadcasted_iota(jnp.int32, (bi, bj), 1))
        tri = j_glob > i_glob

        safe_d2 = jnp.where(tri, d2, 1.0)
        inv_d = lax.rsqrt(safe_d2)
        d = safe_d2 * inv_d

        si = col_ref[:, 3].reshape(bi, 1)
        sj = row_ref[3, :].reshape(1, bj)
        sij = si * sj

        def bl(cc):
            return cc[0] + cc[1] * si + cc[2] * sj + cc[3] * sij

        inv_s = bl(isc)
        a = bl(ac)
        e_a = bl(eoa)
        ne_s = bl(neos)

        braw = 1.0 - d * inv_s
        inside = (d2 < cut2) & (braw > 0.0) & tri
        b = jnp.where(inside, braw, 0.5)
        lnb = jnp.log(b)
        q = jnp.exp((a - 1.0) * lnb)
        p = q * b

        pe = jnp.where(inside, e_a * p, 0.0)
        coeff = jnp.where(inside, ne_s * q * inv_d, 0.0)
        w0 = coeff * dfrac[0]
        w1 = coeff * dfrac[1]
        w2 = coeff * dfrac[2]

        gfx = jnp.sum(w0, axis=1).reshape(bi, 1)
        gfy = jnp.sum(w1, axis=1).reshape(bi, 1)
        gfz = jnp.sum(w2, axis=1).reshape(bi, 1)
        pes = jnp.sum(pe, axis=1).reshape(bi, 1)
        fx = gfx * cl[0][0] + gfy * cl[1][0] + gfz * cl[2][0]
        fy = gfx * cl[0][1] + gfy * cl[1][1] + gfz * cl[2][1]
        fz = gfx * cl[0][2] + gfy * cl[1][2] + gfz * cl[2][2]
        zeros = jnp.zeros((bi, 4), dtype=jnp.float32)
        orow_ref[...] += jnp.concatenate([fx, fy, fz, pes, zeros], axis=1)

        cgx = -jnp.sum(w0, axis=0).reshape(1, bj)
        cgy = -jnp.sum(w1, axis=0).reshape(1, bj)
        cgz = -jnp.sum(w2, axis=0).reshape(1, bj)
        cfx = cgx * cl[0][0] + cgy * cl[1][0] + cgz * cl[2][0]
        cfy = cgx * cl[0][1] + cgy * cl[1][1] + cgz * cl[2][1]
        cfz = cgx * cl[0][2] + cgy * cl[1][2] + cgz * cl[2][2]
        zcol = jnp.zeros((5, bj), dtype=jnp.float32)
        ocol_ref[...] = jnp.concatenate(
            [cfx, cfy, cfz, zcol], axis=0)[None]

    @pl.when(jnp.logical_not(include))
    def _():
        ocol_ref[...] = jnp.zeros_like(ocol_ref)


def _inv3(c):
    a00, a01, a02 = c[0, 0], c[0, 1], c[0, 2]
    a10, a11, a12 = c[1, 0], c[1, 1], c[1, 2]
    a20, a21, a22 = c[2, 0], c[2, 1], c[2, 2]
    c00 = a11 * a22 - a12 * a21
    c01 = a02 * a21 - a01 * a22
    c02 = a01 * a12 - a02 * a11
    c10 = a12 * a20 - a10 * a22
    c11 = a00 * a22 - a02 * a20
    c12 = a02 * a10 - a00 * a12
    c20 = a10 * a21 - a11 * a20
    c21 = a01 * a20 - a00 * a21
    c22 = a00 * a11 - a01 * a10
    det = a00 * c00 + a01 * c10 + a02 * c20
    adj = jnp.stack([jnp.stack([c00, c01, c02]),
                     jnp.stack([c10, c11, c12]),
                     jnp.stack([c20, c21, c22])])
    return adj / det


def kernel(positions, cell, pbc, species, sigma_matrix, epsilon_matrix, alpha_matrix, cutoff):
    n = positions.shape[0]
    inv_cell = _inv3(cell)
    frac = positions @ inv_cell
    spf = species.astype(jnp.float32)

    g = cell @ cell.T
    cf32 = cutoff.astype(jnp.float32)
    params = jnp.zeros((48,), jnp.float32)
    params = params.at[0:3].set(jnp.diag(g).astype(jnp.float32))
    params = params.at[3].set(2.0 * g[0, 1])
    params = params.at[4].set(2.0 * g[0, 2])
    params = params.at[5].set(2.0 * g[1, 2])
    params = params.at[6:9].set(pbc.astype(jnp.float32))
    params = params.at[12].set(cf32 * cf32)
    params = params.at[14:18].set(_bilin(1.0 / sigma_matrix))
    params = params.at[18:22].set(_bilin(alpha_matrix))
    params = params.at[22:26].set(_bilin(epsilon_matrix / alpha_matrix))
    params = params.at[26:30].set(_bilin(-epsilon_matrix / sigma_matrix))
    params = params.at[30:39].set(cell.reshape(9).astype(jnp.float32))

    mesh = plsc.VectorSubcoreMesh(core_axis_name="c", subcore_axis_name="s")
    sc_f = functools.partial(
        pl.kernel,
        mesh=mesh,
        out_type=[jax.ShapeDtypeStruct((_C_SC,), jnp.float32)] * 4,
        scratch_types=[
            pltpu.VMEM((_C_SC,), jnp.float32),
            pltpu.VMEM((_C_SC,), jnp.float32),
            pltpu.VMEM((_C_SC,), jnp.float32),
            pltpu.VMEM((_C_SC,), jnp.float32),
            pltpu.VMEM((48,), jnp.float32),
            pltpu.VMEM((4 * _C_SC // _NW,), jnp.float32),
        ],
    )(_sc_pair_kernel)
    ofx, ofy, ofz, ope = sc_f(frac[:, 0], frac[:, 1], frac[:, 2], spf, params)

    col = jnp.concatenate(
        [frac, spf[:, None], jnp.zeros((n, 4), jnp.float32)], axis=1)
    row = col[_C_SC:].T

    njb = (n - _C_SC) // _BJ
    nib = n // _BI
    orow, ocol = pl.pallas_call(
        _tc_pair_kernel,
        grid=(nib, njb),
        in_specs=[
            pl.BlockSpec(memory_space=pltpu.SMEM),
            pl.BlockSpec((8, _BJ), lambda i, j: (0, j)),
            pl.BlockSpec((_BI, 8), lambda i, j: (i, 0)),
        ],
        out_specs=[
            pl.BlockSpec((_BI, 8), lambda i, j: (i, 0)),
            pl.BlockSpec((1, 8, _BJ), lambda i, j: (i, 0, j)),
        ],
        out_shape=[
            jax.ShapeDtypeStruct((n, 8), jnp.float32),
            jax.ShapeDtypeStruct((nib, 8, n - _C_SC), jnp.float32),
        ],
    )(params, row, col)

    col_sum = jnp.sum(ocol, axis=0)
    sc_force = jnp.stack([ofx, ofy, ofz], axis=1)
    tail = jnp.concatenate([sc_force, col_sum[:3].T], axis=0)
    forces = orow[:, :3] + tail
    energy = 0.5 * jnp.sum(ope) + jnp.sum(orow[:, 3])
    return energy, forces
